# Initial kernel scaffold; baseline (speedup 1.0000x reference)
import jax
import jax.numpy as jnp
from jax.experimental import pallas as pl


def kernel(x, edge_index, W, a_w, bias):
    raise NotImplementedError("write your pallas kernel here")


# speedup vs baseline: 9.5574x; 9.5574x over previous
import functools

import jax
import jax.numpy as jnp
from jax import lax
from jax.experimental import pallas as pl
from jax.experimental.pallas import tpu as pltpu
from jax.experimental.pallas import tpu_sc as plsc

_NC = 2
_NS = 16
_AW = 144


def _tc_prep_body(x_ref, wt_ref, a1_ref, a2_ref, wh_ref, s1_ref, s2_ref):
    wh = jnp.dot(x_ref[...], wt_ref[...], preferred_element_type=jnp.float32)
    wh_ref[...] = wh
    s1_ref[...] = jnp.dot(wh, a1_ref[...], preferred_element_type=jnp.float32)
    s2_ref[...] = jnp.dot(wh, a2_ref[...], preferred_element_type=jnp.float32)


def _tc_prep(x, wt, a1, a2):
    n, d_in = x.shape
    d_out = wt.shape[1]
    blk = 1000
    return pl.pallas_call(
        _tc_prep_body,
        grid=(n // blk,),
        in_specs=[
            pl.BlockSpec((blk, d_in), lambda i: (i, 0)),
            pl.BlockSpec((d_in, d_out), lambda i: (0, 0)),
            pl.BlockSpec((d_out, 1), lambda i: (0, 0)),
            pl.BlockSpec((d_out, 1), lambda i: (0, 0)),
        ],
        out_specs=[
            pl.BlockSpec((blk, d_out), lambda i: (i, 0)),
            pl.BlockSpec((blk, 1), lambda i: (i, 0)),
            pl.BlockSpec((blk, 1), lambda i: (i, 0)),
        ],
        out_shape=[
            jax.ShapeDtypeStruct((n, d_out), jnp.float32),
            jax.ShapeDtypeStruct((n, 1), jnp.float32),
            jax.ShapeDtypeStruct((n, 1), jnp.float32),
        ],
    )(x, wt, a1, a2)


def _sc_agg(wh, src, dst, s1, s2):
    n, d = wh.shape
    e = src.shape[0]
    nw = _NC * _NS
    epw = e // nw
    k = 80
    nchunk = epw // k
    zr = 8
    wr = 200
    nwc = n // wr
    mesh = plsc.VectorSubcoreMesh(core_axis_name="c", subcore_axis_name="s")

    @functools.partial(
        pl.kernel,
        out_type=jax.ShapeDtypeStruct((_NC, n, _AW), jnp.float32),
        mesh=mesh,
        scratch_types=[
            pltpu.VMEM((k,), jnp.int32),
            pltpu.VMEM((k,), jnp.int32),
            pltpu.VMEM((k,), jnp.float32),
            pltpu.VMEM((k,), jnp.float32),
            pltpu.VMEM((k, d), jnp.float32),
            pltpu.VMEM((k, _AW), jnp.float32),
            pltpu.VMEM_SHARED((n, _AW), jnp.float32),
            pltpu.SemaphoreType.DMA,
        ],
        compiler_params=pltpu.CompilerParams(needs_layout_passes=False,
                                             use_tc_tiling_on_sc=False),
    )
    def sc_kernel(wh_hbm, src_hbm, dst_hbm, s1_hbm, s2_hbm, out_hbm,
                  si_v, di_v, s1g_v, s2g_v, rows_v, stg_v,
                  acc_sh, sem):
        c = lax.axis_index("c")
        s = lax.axis_index("s")
        wid = c * _NS + s

        zero16 = jnp.zeros((16,), jnp.float32)

        def zrow(i, carry):
            for j in range(_AW // 16):
                stg_v[i, pl.ds(j * 16, 16)] = zero16
            return carry

        lax.fori_loop(0, k, zrow, 0)

        nzc = n // zr

        def zacc(i, carry):
            cidx = i * _NS + s

            @pl.when(cidx < nzc)
            def _():
                pltpu.sync_copy(stg_v.at[pl.ds(0, zr)],
                                acc_sh.at[pl.ds(cidx * zr, zr)])

            return carry

        lax.fori_loop(0, -(-nzc // _NS), zacc, 0)
        plsc.subcore_barrier()

        lane0 = lax.iota(jnp.int32, 16) == 0

        def _exp(t):
            a = jnp.clip(t, -80.0, 80.0) * 1.4426950408889634
            ni = (a + jnp.where(a >= 0.0, 0.5, -0.5)).astype(jnp.int32)
            r = a - ni.astype(jnp.float32)
            q = r * 0.6931471805599453
            pol = 1.0 + q * (1.0 + q * (0.5 + q * (
                0.16666667 + q * (0.041666668 + q * (
                    0.008333334 + q * 0.0013888889)))))
            scale = plsc.bitcast(lax.shift_left(ni + 127, 23), jnp.float32)
            return pol * scale

        def chunk(i, carry):
            base = wid * epw + i * k
            pltpu.sync_copy(src_hbm.at[pl.ds(base, k)], si_v)
            pltpu.sync_copy(dst_hbm.at[pl.ds(base, k)], di_v)
            cp1 = pltpu.async_copy(s1_hbm.at[si_v], s1g_v, sem)
            cp2 = pltpu.async_copy(s2_hbm.at[di_v], s2g_v, sem)
            cp3 = pltpu.async_copy(wh_hbm.at[si_v], rows_v, sem)
            cp1.wait()
            cp2.wait()
            for g in range(k // 16):
                t = s1g_v[pl.ds(g * 16, 16)] + s2g_v[pl.ds(g * 16, 16)]
                t = jnp.where(t >= 0.0, t, 0.2 * t)
                s1g_v[pl.ds(g * 16, 16)] = _exp(t)
            cp3.wait()

            def edge_group(gi, carry2):
                pg = s1g_v[pl.ds(gi * 16, 16)]
                for l in range(16):
                    ei = gi * 16 + l
                    pe = pg[l]
                    for j in range(d // 16):
                        stg_v[ei, pl.ds(j * 16, 16)] = (
                            rows_v[ei, pl.ds(j * 16, 16)] * pe)
                    stg_v[ei, pl.ds(d, 16)] = jnp.where(lane0, pe, 0.0)
                return carry2

            lax.fori_loop(0, k // 16, edge_group, 0)
            pltpu.sync_copy(stg_v, acc_sh.at[di_v], add=True)
            return carry

        lax.fori_loop(0, nchunk, chunk, 0)
        plsc.subcore_barrier()
        for kk in range(-(-nwc // _NS)):
            cidx = kk * _NS + s

            @pl.when(cidx < nwc)
            def _():
                pltpu.sync_copy(acc_sh.at[pl.ds(cidx * wr, wr)],
                                out_hbm.at[c, pl.ds(cidx * wr, wr)])

    return sc_kernel(wh, src, dst, s1, s2)


def _tc_fin_body(acc_ref, wh_ref, b_ref, o_ref):
    d = wh_ref.shape[1]
    num = acc_ref[0, :, 0:d] + acc_ref[1, :, 0:d]
    den = acc_ref[0, :, d:d + 1] + acc_ref[1, :, d:d + 1]
    h = num / jnp.maximum(den, 1e-9) + wh_ref[...] + b_ref[...]
    o_ref[...] = jnp.maximum(h, 0.0)


def _tc_fin(acc, wh, bias):
    n, d = wh.shape
    blk = 1000
    return pl.pallas_call(
        _tc_fin_body,
        grid=(n // blk,),
        in_specs=[
            pl.BlockSpec((_NC, blk, _AW), lambda i: (0, i, 0)),
            pl.BlockSpec((blk, d), lambda i: (i, 0)),
            pl.BlockSpec((1, d), lambda i: (0, 0)),
        ],
        out_specs=pl.BlockSpec((blk, d), lambda i: (i, 0)),
        out_shape=jax.ShapeDtypeStruct((n, d), jnp.float32),
    )(acc, wh, bias)


def kernel(x, edge_index, W, a_w, bias):
    n = x.shape[0]
    d_out = W.shape[0]
    src = edge_index[0]
    dst = edge_index[1]
    wt = W.T
    a1 = a_w[0, :d_out].reshape(d_out, 1)
    a2 = a_w[0, d_out:].reshape(d_out, 1)
    wh, s1, s2 = _tc_prep(x, wt, a1, a2)
    acc = _sc_agg(wh, src, dst, s1.reshape(n), s2.reshape(n))
    return _tc_fin(acc, wh, bias)


# speedup vs baseline: 22.5943x; 2.3641x over previous
import functools

import jax
import jax.numpy as jnp
from jax import lax
from jax.experimental import pallas as pl
from jax.experimental.pallas import tpu as pltpu
from jax.experimental.pallas import tpu_sc as plsc

_NC = 2
_NS = 16
_AW = 144


def _tc_prep_body(x_ref, wt_ref, a1_ref, a2_ref, wh_ref, s1_ref, s2_ref):
    wh = jnp.dot(x_ref[...], wt_ref[...], preferred_element_type=jnp.float32)
    wh_ref[...] = wh
    s1_ref[...] = jnp.dot(wh, a1_ref[...], preferred_element_type=jnp.float32)
    s2_ref[...] = jnp.dot(wh, a2_ref[...], preferred_element_type=jnp.float32)


def _tc_prep(x, wt, a1, a2):
    n, d_in = x.shape
    d_out = wt.shape[1]
    blk = 1000
    return pl.pallas_call(
        _tc_prep_body,
        grid=(n // blk,),
        in_specs=[
            pl.BlockSpec((blk, d_in), lambda i: (i, 0)),
            pl.BlockSpec((d_in, d_out), lambda i: (0, 0)),
            pl.BlockSpec((d_out, 1), lambda i: (0, 0)),
            pl.BlockSpec((d_out, 1), lambda i: (0, 0)),
        ],
        out_specs=[
            pl.BlockSpec((blk, d_out), lambda i: (i, 0)),
            pl.BlockSpec((blk, 1), lambda i: (i, 0)),
            pl.BlockSpec((blk, 1), lambda i: (i, 0)),
        ],
        out_shape=[
            jax.ShapeDtypeStruct((n, d_out), jnp.float32),
            jax.ShapeDtypeStruct((n, 1), jnp.float32),
            jax.ShapeDtypeStruct((n, 1), jnp.float32),
        ],
    )(x, wt, a1, a2)


def _sc_agg(wh, src, dst, s1, s2):
    n, d = wh.shape
    e = src.shape[0]
    nw = _NC * _NS
    epw = e // nw
    k = 80
    nchunk = epw // k
    npair = (nchunk - 1) // 2
    zr = 80
    wr = 200
    nwc = n // wr
    mesh = plsc.VectorSubcoreMesh(core_axis_name="c", subcore_axis_name="s")

    @functools.partial(
        pl.kernel,
        out_type=(jax.ShapeDtypeStruct((_NC, n, d), jnp.float32),
                  jax.ShapeDtypeStruct((_NC, n, 16), jnp.float32)),
        mesh=mesh,
        scratch_types=[
            pltpu.VMEM((k,), jnp.int32),
            pltpu.VMEM((k,), jnp.int32),
            pltpu.VMEM((k,), jnp.int32),
            pltpu.VMEM((k,), jnp.int32),
            pltpu.VMEM((k,), jnp.float32),
            pltpu.VMEM((k,), jnp.float32),
            pltpu.VMEM((k,), jnp.float32),
            pltpu.VMEM((k,), jnp.float32),
            pltpu.VMEM((k, d), jnp.float32),
            pltpu.VMEM((k, d), jnp.float32),
            pltpu.VMEM((k, 16), jnp.float32),
            pltpu.VMEM((k, 16), jnp.float32),
            pltpu.VMEM_SHARED((n, d), jnp.float32),
            pltpu.VMEM_SHARED((n, 16), jnp.float32),
            pltpu.SemaphoreType.DMA,
            pltpu.SemaphoreType.DMA,
            pltpu.SemaphoreType.DMA,
            pltpu.SemaphoreType.DMA,
        ],
        compiler_params=pltpu.CompilerParams(needs_layout_passes=False,
                                             use_tc_tiling_on_sc=False),
    )
    def sc_kernel(wh_hbm, src_hbm, dst_hbm, s1_hbm, s2_hbm,
                  onum_hbm, oden_hbm,
                  si0_v, si1_v, di0_v, di1_v, s1g0_v, s1g1_v,
                  s2g0_v, s2g1_v, rows0_v, rows1_v,
                  pd0_v, pd1_v, anum_sh, aden_sh,
                  gsem0, gsem1, ssem0, ssem1):
        c = lax.axis_index("c")
        s = lax.axis_index("s")
        wid = c * _NS + s
        si_b = (si0_v, si1_v)
        di_b = (di0_v, di1_v)
        s1g_b = (s1g0_v, s1g1_v)
        s2g_b = (s2g0_v, s2g1_v)
        rows_b = (rows0_v, rows1_v)
        pd_b = (pd0_v, pd1_v)
        gsem_b = (gsem0, gsem1)
        ssem_b = (ssem0, ssem1)

        zero16 = jnp.zeros((16,), jnp.float32)

        def zrow(i, carry):
            for j in range(d // 16):
                rows0_v[i, pl.ds(j * 16, 16)] = zero16
            pd0_v[i, pl.ds(0, 16)] = zero16
            return carry

        lax.fori_loop(0, k, zrow, 0)

        nzc = n // zr

        def zacc(i, carry):
            cidx = i * _NS + s

            @pl.when(cidx < nzc)
            def _():
                pltpu.sync_copy(rows0_v, anum_sh.at[pl.ds(cidx * zr, zr)])
                pltpu.sync_copy(pd0_v, aden_sh.at[pl.ds(cidx * zr, zr)])

            return carry

        lax.fori_loop(0, -(-nzc // _NS), zacc, 0)
        plsc.subcore_barrier()

        lane0 = lax.iota(jnp.int32, 16) == 0
        ebase = wid * epw

        def _exp(t):
            a = jnp.clip(t, -80.0, 80.0) * 1.4426950408889634
            ni = (a + jnp.where(a >= 0.0, 0.5, -0.5)).astype(jnp.int32)
            r = a - ni.astype(jnp.float32)
            q = r * 0.6931471805599453
            pol = 1.0 + q * (1.0 + q * (0.5 + q * (
                0.16666667 + q * (0.041666668 + q * (
                    0.008333334 + q * 0.0013888889)))))
            scale = plsc.bitcast(lax.shift_left(ni + 127, 23), jnp.float32)
            return pol * scale

        def load_idx(ci, b):
            pltpu.sync_copy(src_hbm.at[pl.ds(ebase + ci * k, k)], si_b[b])
            pltpu.sync_copy(dst_hbm.at[pl.ds(ebase + ci * k, k)], di_b[b])

        def fire_gathers(b):
            pltpu.async_copy(s1_hbm.at[si_b[b]], s1g_b[b], gsem_b[b])
            pltpu.async_copy(s2_hbm.at[di_b[b]], s2g_b[b], gsem_b[b])
            pltpu.async_copy(wh_hbm.at[si_b[b]], rows_b[b], gsem_b[b])

        def wait_gathers(b):
            pltpu.make_async_copy(s1_hbm.at[si_b[b]], s1g_b[b],
                                  gsem_b[b]).wait()
            pltpu.make_async_copy(s2_hbm.at[di_b[b]], s2g_b[b],
                                  gsem_b[b]).wait()
            pltpu.make_async_copy(wh_hbm.at[si_b[b]], rows_b[b],
                                  gsem_b[b]).wait()

        def fire_scatter(b):
            pltpu.async_copy(rows_b[b], anum_sh.at[di_b[b]], ssem_b[b],
                             add=True)
            pltpu.async_copy(pd_b[b], aden_sh.at[di_b[b]], ssem_b[b],
                             add=True)

        def wait_scatter(b):
            pltpu.make_async_copy(rows_b[b], anum_sh.at[di_b[b]],
                                  ssem_b[b]).wait()
            pltpu.make_async_copy(pd_b[b], aden_sh.at[di_b[b]],
                                  ssem_b[b]).wait()

        def compute(b):
            rows = rows_b[b]
            pd = pd_b[b]
            s1g = s1g_b[b]
            s2g = s2g_b[b]
            for g in range(k // 16):
                t = s1g[pl.ds(g * 16, 16)] + s2g[pl.ds(g * 16, 16)]
                t = jnp.where(t >= 0.0, t, 0.2 * t)
                s1g[pl.ds(g * 16, 16)] = _exp(t)

            def edge_group(gi, carry2):
                pg = s1g[pl.ds(gi * 16, 16)]
                for l in range(16):
                    ei = gi * 16 + l
                    pe = pg[l]
                    for j in range(d // 16):
                        rows[ei, pl.ds(j * 16, 16)] = (
                            rows[ei, pl.ds(j * 16, 16)] * pe)
                    pd[ei, pl.ds(0, 16)] = jnp.where(lane0, pe, 0.0)
                return carry2

            lax.fori_loop(0, k // 16, edge_group, 0)

        load_idx(0, 0)
        fire_gathers(0)

        def pair(j, carry):
            c0 = 2 * j

            @pl.when(j > 0)
            def _():
                wait_scatter(1)

            load_idx(c0 + 1, 1)
            fire_gathers(1)
            wait_gathers(0)
            compute(0)
            fire_scatter(0)
            wait_gathers(1)
            compute(1)
            fire_scatter(1)
            wait_scatter(0)
            load_idx(c0 + 2, 0)
            fire_gathers(0)
            return carry

        lax.fori_loop(0, npair, pair, 0)
        wait_scatter(1)
        wait_gathers(0)
        compute(0)
        fire_scatter(0)
        wait_scatter(0)
        plsc.subcore_barrier()
        for kk in range(-(-nwc // _NS)):
            cidx = kk * _NS + s

            @pl.when(cidx < nwc)
            def _():
                pltpu.sync_copy(anum_sh.at[pl.ds(cidx * wr, wr)],
                                onum_hbm.at[c, pl.ds(cidx * wr, wr)])
                pltpu.sync_copy(aden_sh.at[pl.ds(cidx * wr, wr)],
                                oden_hbm.at[c, pl.ds(cidx * wr, wr)])

    return sc_kernel(wh, src, dst, s1, s2)


def _tc_fin_body(num_ref, den_ref, wh_ref, b_ref, o_ref):
    num = num_ref[0] + num_ref[1]
    den = den_ref[0, :, 0:1] + den_ref[1, :, 0:1]
    h = num / jnp.maximum(den, 1e-9) + wh_ref[...] + b_ref[...]
    o_ref[...] = jnp.maximum(h, 0.0)


def _tc_fin(acc_num, acc_den, wh, bias):
    n, d = wh.shape
    blk = 1000
    return pl.pallas_call(
        _tc_fin_body,
        grid=(n // blk,),
        in_specs=[
            pl.BlockSpec((_NC, blk, d), lambda i: (0, i, 0)),
            pl.BlockSpec((_NC, blk, 16), lambda i: (0, i, 0)),
            pl.BlockSpec((blk, d), lambda i: (i, 0)),
            pl.BlockSpec((1, d), lambda i: (0, 0)),
        ],
        out_specs=pl.BlockSpec((blk, d), lambda i: (i, 0)),
        out_shape=jax.ShapeDtypeStruct((n, d), jnp.float32),
    )(acc_num, acc_den, wh, bias)


def kernel(x, edge_index, W, a_w, bias):
    n = x.shape[0]
    d_out = W.shape[0]
    src = edge_index[0]
    dst = edge_index[1]
    wt = W.T
    a1 = a_w[0, :d_out].reshape(d_out, 1)
    a2 = a_w[0, d_out:].reshape(d_out, 1)
    wh, s1, s2 = _tc_prep(x, wt, a1, a2)
    acc_num, acc_den = _sc_agg(wh, src, dst, s1.reshape(n), s2.reshape(n))
    return _tc_fin(acc_num, acc_den, wh, bias)


# speedup vs baseline: 27.4999x; 1.2171x over previous
import functools

import jax
import jax.numpy as jnp
from jax import lax
from jax.experimental import pallas as pl
from jax.experimental.pallas import tpu as pltpu
from jax.experimental.pallas import tpu_sc as plsc

_NC = 2
_NS = 16
_AW = 144


def _tc_prep_body(x_ref, wt_ref, a1_ref, a2_ref, wh_ref, s1_ref, s2_ref):
    wh = jnp.dot(x_ref[...], wt_ref[...], preferred_element_type=jnp.float32)
    wh_ref[...] = wh
    s1_ref[...] = jnp.dot(wh, a1_ref[...], preferred_element_type=jnp.float32)
    s2_ref[...] = jnp.dot(wh, a2_ref[...], preferred_element_type=jnp.float32)


def _tc_prep(x, wt, a1, a2):
    n, d_in = x.shape
    d_out = wt.shape[1]
    blk = 1000
    return pl.pallas_call(
        _tc_prep_body,
        grid=(n // blk,),
        in_specs=[
            pl.BlockSpec((blk, d_in), lambda i: (i, 0)),
            pl.BlockSpec((d_in, d_out), lambda i: (0, 0)),
            pl.BlockSpec((d_out, 1), lambda i: (0, 0)),
            pl.BlockSpec((d_out, 1), lambda i: (0, 0)),
        ],
        out_specs=[
            pl.BlockSpec((blk, d_out), lambda i: (i, 0)),
            pl.BlockSpec((blk, 1), lambda i: (i, 0)),
            pl.BlockSpec((blk, 1), lambda i: (i, 0)),
        ],
        out_shape=[
            jax.ShapeDtypeStruct((n, d_out), jnp.float32),
            jax.ShapeDtypeStruct((n, 1), jnp.float32),
            jax.ShapeDtypeStruct((n, 1), jnp.float32),
        ],
    )(x, wt, a1, a2)


def _sc_agg(wh, src, dst, s1, s2):
    n, d = wh.shape
    e = src.shape[0]
    nw = _NC * _NS
    epw = e // nw
    k = 80
    nchunk = epw // k
    npair = (nchunk - 1) // 2
    zr = 80
    wr = 200
    nwc = n // wr
    mesh = plsc.VectorSubcoreMesh(core_axis_name="c", subcore_axis_name="s")

    @functools.partial(
        pl.kernel,
        out_type=(jax.ShapeDtypeStruct((_NC, n, d), jnp.float32),
                  jax.ShapeDtypeStruct((_NC, n, 16), jnp.float32)),
        mesh=mesh,
        scratch_types=[
            pltpu.VMEM((k,), jnp.int32),
            pltpu.VMEM((k,), jnp.int32),
            pltpu.VMEM((k,), jnp.int32),
            pltpu.VMEM((k,), jnp.int32),
            pltpu.VMEM((k,), jnp.int32),
            pltpu.VMEM((k,), jnp.int32),
            pltpu.VMEM((k,), jnp.float32),
            pltpu.VMEM((k,), jnp.float32),
            pltpu.VMEM((k,), jnp.float32),
            pltpu.VMEM((k,), jnp.float32),
            pltpu.VMEM((k, d), jnp.float32),
            pltpu.VMEM((k, d), jnp.float32),
            pltpu.VMEM((k, 16), jnp.float32),
            pltpu.VMEM((k, 16), jnp.float32),
            pltpu.VMEM_SHARED((n, d), jnp.float32),
            pltpu.VMEM_SHARED((n, 16), jnp.float32),
            pltpu.SemaphoreType.DMA,
            pltpu.SemaphoreType.DMA,
            pltpu.SemaphoreType.DMA,
            pltpu.SemaphoreType.DMA,
            pltpu.SemaphoreType.DMA,
            pltpu.SemaphoreType.DMA,
        ],
        compiler_params=pltpu.CompilerParams(needs_layout_passes=False,
                                             use_tc_tiling_on_sc=False),
    )
    def sc_kernel(wh_hbm, src_hbm, dst_hbm, s1_hbm, s2_hbm,
                  onum_hbm, oden_hbm,
                  si0_v, si1_v, di0_v, di1_v, dis0_v, dis1_v,
                  s1g0_v, s1g1_v, s2g0_v, s2g1_v, rows0_v, rows1_v,
                  pd0_v, pd1_v, anum_sh, aden_sh,
                  gsem0, gsem1, ssem0, ssem1, isem0, isem1):
        c = lax.axis_index("c")
        s = lax.axis_index("s")
        wid = c * _NS + s
        si_b = (si0_v, si1_v)
        di_b = (di0_v, di1_v)
        dis_b = (dis0_v, dis1_v)
        isem_b = (isem0, isem1)
        s1g_b = (s1g0_v, s1g1_v)
        s2g_b = (s2g0_v, s2g1_v)
        rows_b = (rows0_v, rows1_v)
        pd_b = (pd0_v, pd1_v)
        gsem_b = (gsem0, gsem1)
        ssem_b = (ssem0, ssem1)

        zero16 = jnp.zeros((16,), jnp.float32)

        def zrow(i, carry):
            for j in range(d // 16):
                rows0_v[i, pl.ds(j * 16, 16)] = zero16
            pd0_v[i, pl.ds(0, 16)] = zero16
            return carry

        lax.fori_loop(0, k, zrow, 0)

        nzc = n // zr

        def zacc(i, carry):
            cidx = i * _NS + s

            @pl.when(cidx < nzc)
            def _():
                pltpu.sync_copy(rows0_v, anum_sh.at[pl.ds(cidx * zr, zr)])
                pltpu.sync_copy(pd0_v, aden_sh.at[pl.ds(cidx * zr, zr)])

            return carry

        lax.fori_loop(0, -(-nzc // _NS), zacc, 0)
        plsc.subcore_barrier()

        lane0 = lax.iota(jnp.int32, 16) == 0
        ebase = wid * epw

        def _exp(t):
            a = jnp.clip(t, -80.0, 80.0) * 1.4426950408889634
            ni = (a + jnp.where(a >= 0.0, 0.5, -0.5)).astype(jnp.int32)
            r = a - ni.astype(jnp.float32)
            q = r * 0.6931471805599453
            pol = 1.0 + q * (1.0 + q * (0.5 + q * (
                0.16666667 + q * (0.041666668 + q * (
                    0.008333334 + q * 0.0013888889)))))
            scale = plsc.bitcast(lax.shift_left(ni + 127, 23), jnp.float32)
            return pol * scale

        def fire_idx(ci, b):
            pltpu.async_copy(src_hbm.at[pl.ds(ebase + ci * k, k)],
                             si_b[b], isem_b[b])
            pltpu.async_copy(dst_hbm.at[pl.ds(ebase + ci * k, k)],
                             di_b[b], isem_b[b])

        def wait_idx(ci, b):
            pltpu.make_async_copy(src_hbm.at[pl.ds(ebase + ci * k, k)],
                                  si_b[b], isem_b[b]).wait()
            pltpu.make_async_copy(dst_hbm.at[pl.ds(ebase + ci * k, k)],
                                  di_b[b], isem_b[b]).wait()

        def fire_gathers(b):
            pltpu.async_copy(s1_hbm.at[si_b[b]], s1g_b[b], gsem_b[b])
            pltpu.async_copy(s2_hbm.at[di_b[b]], s2g_b[b], gsem_b[b])
            pltpu.async_copy(wh_hbm.at[si_b[b]], rows_b[b], gsem_b[b])

        def wait_gathers(b):
            pltpu.make_async_copy(s1_hbm.at[si_b[b]], s1g_b[b],
                                  gsem_b[b]).wait()
            pltpu.make_async_copy(s2_hbm.at[di_b[b]], s2g_b[b],
                                  gsem_b[b]).wait()
            pltpu.make_async_copy(wh_hbm.at[si_b[b]], rows_b[b],
                                  gsem_b[b]).wait()

        def fire_scatter(b):
            for g in range(k // 16):
                dis_b[b][pl.ds(g * 16, 16)] = di_b[b][pl.ds(g * 16, 16)]
            pltpu.async_copy(rows_b[b], anum_sh.at[dis_b[b]], ssem_b[b],
                             add=True)
            pltpu.async_copy(pd_b[b], aden_sh.at[dis_b[b]], ssem_b[b],
                             add=True)

        def wait_scatter(b):
            pltpu.make_async_copy(rows_b[b], anum_sh.at[dis_b[b]],
                                  ssem_b[b]).wait()
            pltpu.make_async_copy(pd_b[b], aden_sh.at[dis_b[b]],
                                  ssem_b[b]).wait()

        def compute(b):
            rows = rows_b[b]
            pd = pd_b[b]
            s1g = s1g_b[b]
            s2g = s2g_b[b]
            for g in range(k // 16):
                t = s1g[pl.ds(g * 16, 16)] + s2g[pl.ds(g * 16, 16)]
                t = jnp.where(t >= 0.0, t, 0.2 * t)
                s1g[pl.ds(g * 16, 16)] = _exp(t)

            def edge_group(gi, carry2):
                pg = s1g[pl.ds(gi * 16, 16)]
                for l in range(16):
                    ei = gi * 16 + l
                    pe = pg[l]
                    for j in range(d // 16):
                        rows[ei, pl.ds(j * 16, 16)] = (
                            rows[ei, pl.ds(j * 16, 16)] * pe)
                    pd[ei, pl.ds(0, 16)] = jnp.where(lane0, pe, 0.0)
                return carry2

            lax.fori_loop(0, k // 16, edge_group, 0)

        fire_idx(0, 0)
        fire_idx(1, 1)
        wait_idx(0, 0)
        fire_gathers(0)

        def pair(j, carry):
            c0 = 2 * j
            wait_idx(c0 + 1, 1)

            @pl.when(j > 0)
            def _():
                wait_scatter(1)

            fire_gathers(1)
            wait_gathers(0)
            compute(0)
            fire_scatter(0)
            fire_idx(c0 + 2, 0)
            wait_gathers(1)
            compute(1)
            fire_scatter(1)

            @pl.when(c0 + 3 < nchunk)
            def _():
                fire_idx(c0 + 3, 1)

            wait_idx(c0 + 2, 0)
            wait_scatter(0)
            fire_gathers(0)
            return carry

        lax.fori_loop(0, npair, pair, 0)
        wait_scatter(1)
        wait_gathers(0)
        compute(0)
        fire_scatter(0)
        wait_scatter(0)
        plsc.subcore_barrier()
        for kk in range(-(-nwc // _NS)):
            cidx = kk * _NS + s

            @pl.when(cidx < nwc)
            def _():
                pltpu.sync_copy(anum_sh.at[pl.ds(cidx * wr, wr)],
                                onum_hbm.at[c, pl.ds(cidx * wr, wr)])
                pltpu.sync_copy(aden_sh.at[pl.ds(cidx * wr, wr)],
                                oden_hbm.at[c, pl.ds(cidx * wr, wr)])

    return sc_kernel(wh, src, dst, s1, s2)


def _tc_fin_body(num_ref, den_ref, wh_ref, b_ref, o_ref):
    num = num_ref[0] + num_ref[1]
    den = den_ref[0, :, 0:1] + den_ref[1, :, 0:1]
    h = num / jnp.maximum(den, 1e-9) + wh_ref[...] + b_ref[...]
    o_ref[...] = jnp.maximum(h, 0.0)


def _tc_fin(acc_num, acc_den, wh, bias):
    n, d = wh.shape
    blk = 1000
    return pl.pallas_call(
        _tc_fin_body,
        grid=(n // blk,),
        in_specs=[
            pl.BlockSpec((_NC, blk, d), lambda i: (0, i, 0)),
            pl.BlockSpec((_NC, blk, 16), lambda i: (0, i, 0)),
            pl.BlockSpec((blk, d), lambda i: (i, 0)),
            pl.BlockSpec((1, d), lambda i: (0, 0)),
        ],
        out_specs=pl.BlockSpec((blk, d), lambda i: (i, 0)),
        out_shape=jax.ShapeDtypeStruct((n, d), jnp.float32),
    )(acc_num, acc_den, wh, bias)


def kernel(x, edge_index, W, a_w, bias):
    n = x.shape[0]
    d_out = W.shape[0]
    src = edge_index[0]
    dst = edge_index[1]
    wt = W.T
    a1 = a_w[0, :d_out].reshape(d_out, 1)
    a2 = a_w[0, d_out:].reshape(d_out, 1)
    wh, s1, s2 = _tc_prep(x, wt, a1, a2)
    acc_num, acc_den = _sc_agg(wh, src, dst, s1.reshape(n), s2.reshape(n))
    return _tc_fin(acc_num, acc_den, wh, bias)


# speedup vs baseline: 32.3709x; 1.1771x over previous
import functools

import jax
import jax.numpy as jnp
from jax import lax
from jax.experimental import pallas as pl
from jax.experimental.pallas import tpu as pltpu
from jax.experimental.pallas import tpu_sc as plsc

_NC = 2
_NS = 16
_AW = 144


def _tc_prep_body(x_ref, wt_ref, a1_ref, a2_ref, wh_ref, s1_ref, s2_ref):
    wh = jnp.dot(x_ref[...], wt_ref[...], preferred_element_type=jnp.float32)
    wh_ref[...] = wh
    s1_ref[...] = jnp.dot(wh, a1_ref[...], preferred_element_type=jnp.float32)
    s2_ref[...] = jnp.dot(wh, a2_ref[...], preferred_element_type=jnp.float32)


def _tc_prep(x, wt, a1, a2):
    n, d_in = x.shape
    d_out = wt.shape[1]
    blk = 1000
    return pl.pallas_call(
        _tc_prep_body,
        grid=(n // blk,),
        in_specs=[
            pl.BlockSpec((blk, d_in), lambda i: (i, 0)),
            pl.BlockSpec((d_in, d_out), lambda i: (0, 0)),
            pl.BlockSpec((d_out, 1), lambda i: (0, 0)),
            pl.BlockSpec((d_out, 1), lambda i: (0, 0)),
        ],
        out_specs=[
            pl.BlockSpec((blk, d_out), lambda i: (i, 0)),
            pl.BlockSpec((blk, 1), lambda i: (i, 0)),
            pl.BlockSpec((blk, 1), lambda i: (i, 0)),
        ],
        out_shape=[
            jax.ShapeDtypeStruct((n, d_out), jnp.float32),
            jax.ShapeDtypeStruct((n, 1), jnp.float32),
            jax.ShapeDtypeStruct((n, 1), jnp.float32),
        ],
    )(x, wt, a1, a2)


def _sc_agg(wh, src, dst, s1, s2):
    n, d = wh.shape
    e = src.shape[0]
    nw = _NC * _NS
    epw = e // nw
    k = 80
    nchunk = epw // k
    npair = (nchunk - 1) // 2
    zr = 80
    wr = 200
    nwc = n // wr
    mesh = plsc.VectorSubcoreMesh(core_axis_name="c", subcore_axis_name="s")

    @functools.partial(
        pl.kernel,
        out_type=(jax.ShapeDtypeStruct((_NC, n, d), jnp.float32),
                  jax.ShapeDtypeStruct((_NC, n, 16), jnp.float32)),
        mesh=mesh,
        scratch_types=[] + [pltpu.VMEM((k,), jnp.int32)] * 9
          + [pltpu.VMEM((k,), jnp.float32)] * 6
          + [pltpu.VMEM((k, d), jnp.float32)] * 3
          + [pltpu.VMEM((k, 16), jnp.float32)] * 3
          + [pltpu.VMEM_SHARED((n, d), jnp.float32),
             pltpu.VMEM_SHARED((n, 16), jnp.float32)]
          + [pltpu.SemaphoreType.DMA] * 9,

        compiler_params=pltpu.CompilerParams(needs_layout_passes=False,
                                             use_tc_tiling_on_sc=False),
    )
    def sc_kernel(wh_hbm, src_hbm, dst_hbm, s1_hbm, s2_hbm,
                  onum_hbm, oden_hbm,
                  si0_v, si1_v, si2_v, di0_v, di1_v, di2_v,
                  dis0_v, dis1_v, dis2_v,
                  s1g0_v, s1g1_v, s1g2_v, s2g0_v, s2g1_v, s2g2_v,
                  rows0_v, rows1_v, rows2_v, pd0_v, pd1_v, pd2_v,
                  anum_sh, aden_sh,
                  gsem0, gsem1, gsem2, ssem0, ssem1, ssem2,
                  isem0, isem1, isem2):
        c = lax.axis_index("c")
        s = lax.axis_index("s")
        wid = c * _NS + s
        si_b = (si0_v, si1_v, si2_v)
        di_b = (di0_v, di1_v, di2_v)
        dis_b = (dis0_v, dis1_v, dis2_v)
        isem_b = (isem0, isem1, isem2)
        s1g_b = (s1g0_v, s1g1_v, s1g2_v)
        s2g_b = (s2g0_v, s2g1_v, s2g2_v)
        rows_b = (rows0_v, rows1_v, rows2_v)
        pd_b = (pd0_v, pd1_v, pd2_v)
        gsem_b = (gsem0, gsem1, gsem2)
        ssem_b = (ssem0, ssem1, ssem2)

        zero16 = jnp.zeros((16,), jnp.float32)

        def zrow(i, carry):
            for j in range(d // 16):
                rows0_v[i, pl.ds(j * 16, 16)] = zero16
            pd0_v[i, pl.ds(0, 16)] = zero16
            return carry

        lax.fori_loop(0, k, zrow, 0)

        nzc = n // zr

        def zacc(i, carry):
            cidx = i * _NS + s

            @pl.when(cidx < nzc)
            def _():
                pltpu.sync_copy(rows0_v, anum_sh.at[pl.ds(cidx * zr, zr)])
                pltpu.sync_copy(pd0_v, aden_sh.at[pl.ds(cidx * zr, zr)])

            return carry

        lax.fori_loop(0, -(-nzc // _NS), zacc, 0)
        plsc.subcore_barrier()

        lane0 = lax.iota(jnp.int32, 16) == 0
        ebase = wid * epw

        def _exp(t):
            a = jnp.clip(t, -80.0, 80.0) * 1.4426950408889634
            ni = (a + jnp.where(a >= 0.0, 0.5, -0.5)).astype(jnp.int32)
            r = a - ni.astype(jnp.float32)
            q = r * 0.6931471805599453
            pol = 1.0 + q * (1.0 + q * (0.5 + q * (
                0.16666667 + q * (0.041666668 + q * (
                    0.008333334 + q * 0.0013888889)))))
            scale = plsc.bitcast(lax.shift_left(ni + 127, 23), jnp.float32)
            return pol * scale

        def fire_idx(ci, b):
            pltpu.async_copy(src_hbm.at[pl.ds(ebase + ci * k, k)],
                             si_b[b], isem_b[b])
            pltpu.async_copy(dst_hbm.at[pl.ds(ebase + ci * k, k)],
                             di_b[b], isem_b[b])

        def wait_idx(ci, b):
            pltpu.make_async_copy(src_hbm.at[pl.ds(ebase + ci * k, k)],
                                  si_b[b], isem_b[b]).wait()
            pltpu.make_async_copy(dst_hbm.at[pl.ds(ebase + ci * k, k)],
                                  di_b[b], isem_b[b]).wait()

        def fire_gathers(ci, b):
            pltpu.async_copy(s1_hbm.at[si_b[b]], s1g_b[b], gsem_b[b])
            pltpu.async_copy(s2_hbm.at[di_b[b]], s2g_b[b], gsem_b[b])
            pltpu.async_copy(wh_hbm.at[si_b[b]], rows_b[b], gsem_b[b])

        def wait_gathers(b):
            pltpu.make_async_copy(s1_hbm.at[si_b[b]], s1g_b[b],
                                  gsem_b[b]).wait()
            pltpu.make_async_copy(s2_hbm.at[di_b[b]], s2g_b[b],
                                  gsem_b[b]).wait()
            pltpu.make_async_copy(wh_hbm.at[si_b[b]], rows_b[b],
                                  gsem_b[b]).wait()

        def fire_scatter(b):
            for g in range(k // 16):
                dis_b[b][pl.ds(g * 16, 16)] = di_b[b][pl.ds(g * 16, 16)]
            pltpu.async_copy(rows_b[b], anum_sh.at[dis_b[b]], ssem_b[b],
                             add=True)
            pltpu.async_copy(pd_b[b], aden_sh.at[dis_b[b]], ssem_b[b],
                             add=True)

        def wait_scatter(b):
            pltpu.make_async_copy(rows_b[b], anum_sh.at[dis_b[b]],
                                  ssem_b[b]).wait()
            pltpu.make_async_copy(pd_b[b], aden_sh.at[dis_b[b]],
                                  ssem_b[b]).wait()

        def compute(b):
            rows = rows_b[b]
            pd = pd_b[b]
            s1g = s1g_b[b]
            s2g = s2g_b[b]
            for g in range(k // 16):
                t = s1g[pl.ds(g * 16, 16)] + s2g[pl.ds(g * 16, 16)]
                t = jnp.where(t >= 0.0, t, 0.2 * t)
                s1g[pl.ds(g * 16, 16)] = _exp(t)

            def edge_group(gi, carry2):
                pg = s1g[pl.ds(gi * 16, 16)]
                for l in range(16):
                    ei = gi * 16 + l
                    pe = pg[l]
                    for j in range(d // 16):
                        rows[ei, pl.ds(j * 16, 16)] = (
                            rows[ei, pl.ds(j * 16, 16)] * pe)
                    pd[ei, pl.ds(0, 16)] = jnp.where(lane0, pe, 0.0)
                return carry2

            lax.fori_loop(0, k // 16, edge_group, 0)

        fire_idx(0, 0)
        fire_idx(1, 1)
        fire_idx(2, 2)
        wait_idx(0, 0)
        fire_gathers(0, 0)
        wait_idx(1, 1)
        fire_gathers(1, 1)

        def body(ci, t, first):
            b2 = (t + 2) % 3
            wait_gathers(t)
            compute(t)
            fire_scatter(t)

            @pl.when(ci + 3 < nchunk)
            def _():
                fire_idx(ci + 3, t)

            @pl.when(ci + 2 < nchunk)
            def _():
                wait_idx(ci + 2, b2)
                if not first:
                    wait_scatter(b2)
                fire_gathers(ci + 2, b2)

        def triple(j, carry):
            for t in range(3):
                body(3 * j + t, t, False)
            return carry

        body(0, 0, True)
        body(1, 1, False)
        body(2, 2, False)
        lax.fori_loop(1, nchunk // 3, triple, 0)
        wait_gathers(0)
        compute(0)
        fire_scatter(0)
        wait_gathers(1)
        compute(1)
        fire_scatter(1)
        wait_scatter(0)
        wait_scatter(1)
        wait_scatter(2)
        plsc.subcore_barrier()
        for kk in range(-(-nwc // _NS)):
            cidx = kk * _NS + s

            @pl.when(cidx < nwc)
            def _():
                pltpu.sync_copy(anum_sh.at[pl.ds(cidx * wr, wr)],
                                onum_hbm.at[c, pl.ds(cidx * wr, wr)])
                pltpu.sync_copy(aden_sh.at[pl.ds(cidx * wr, wr)],
                                oden_hbm.at[c, pl.ds(cidx * wr, wr)])

    return sc_kernel(wh, src, dst, s1, s2)


def _tc_fin_body(num_ref, den_ref, wh_ref, b_ref, o_ref):
    num = num_ref[0] + num_ref[1]
    den = den_ref[0, :, 0:1] + den_ref[1, :, 0:1]
    h = num / jnp.maximum(den, 1e-9) + wh_ref[...] + b_ref[...]
    o_ref[...] = jnp.maximum(h, 0.0)


def _tc_fin(acc_num, acc_den, wh, bias):
    n, d = wh.shape
    blk = 1000
    return pl.pallas_call(
        _tc_fin_body,
        grid=(n // blk,),
        in_specs=[
            pl.BlockSpec((_NC, blk, d), lambda i: (0, i, 0)),
            pl.BlockSpec((_NC, blk, 16), lambda i: (0, i, 0)),
            pl.BlockSpec((blk, d), lambda i: (i, 0)),
            pl.BlockSpec((1, d), lambda i: (0, 0)),
        ],
        out_specs=pl.BlockSpec((blk, d), lambda i: (i, 0)),
        out_shape=jax.ShapeDtypeStruct((n, d), jnp.float32),
    )(acc_num, acc_den, wh, bias)


def kernel(x, edge_index, W, a_w, bias):
    n = x.shape[0]
    d_out = W.shape[0]
    src = edge_index[0]
    dst = edge_index[1]
    wt = W.T
    a1 = a_w[0, :d_out].reshape(d_out, 1)
    a2 = a_w[0, d_out:].reshape(d_out, 1)
    wh, s1, s2 = _tc_prep(x, wt, a1, a2)
    acc_num, acc_den = _sc_agg(wh, src, dst, s1.reshape(n), s2.reshape(n))
    return _tc_fin(acc_num, acc_den, wh, bias)


# speedup vs baseline: 34.3392x; 1.0608x over previous
import functools

import jax
import jax.numpy as jnp
from jax import lax
from jax.experimental import pallas as pl
from jax.experimental.pallas import tpu as pltpu
from jax.experimental.pallas import tpu_sc as plsc

_NC = 2
_NS = 16
_AW = 144


def _tc_prep_body(x_ref, w_ref, a_ref, wh_ref, s1_ref, s2_ref):
    dn = (((1,), (1,)), ((), ()))
    d = w_ref.shape[0]
    wh = lax.dot_general(x_ref[...], w_ref[...], dn,
                         preferred_element_type=jnp.float32)
    wh_ref[...] = wh
    s1_ref[...] = lax.dot_general(wh, a_ref[:, 0:d], dn,
                                  preferred_element_type=jnp.float32)
    s2_ref[...] = lax.dot_general(wh, a_ref[:, d:2 * d], dn,
                                  preferred_element_type=jnp.float32)


def _tc_prep(x, w, a_w):
    n, d_in = x.shape
    d_out = w.shape[0]
    blk = 1000
    return pl.pallas_call(
        _tc_prep_body,
        grid=(n // blk,),
        in_specs=[
            pl.BlockSpec((blk, d_in), lambda i: (i, 0)),
            pl.BlockSpec((d_out, d_in), lambda i: (0, 0)),
            pl.BlockSpec((1, 2 * d_out), lambda i: (0, 0)),
        ],
        out_specs=[
            pl.BlockSpec((blk, d_out), lambda i: (i, 0)),
            pl.BlockSpec((blk, 1), lambda i: (i, 0)),
            pl.BlockSpec((blk, 1), lambda i: (i, 0)),
        ],
        out_shape=[
            jax.ShapeDtypeStruct((n, d_out), jnp.float32),
            jax.ShapeDtypeStruct((n, 1), jnp.float32),
            jax.ShapeDtypeStruct((n, 1), jnp.float32),
        ],
    )(x, w, a_w)


def _sc_agg(wh, edge_index, s1, s2):
    n, d = wh.shape
    e = edge_index.shape[1]
    nw = _NC * _NS
    epw = e // nw
    k = 80
    nchunk = epw // k
    npair = (nchunk - 1) // 2
    zr = 80
    wr = 200
    nwc = n // wr
    mesh = plsc.VectorSubcoreMesh(core_axis_name="c", subcore_axis_name="s")

    @functools.partial(
        pl.kernel,
        out_type=(jax.ShapeDtypeStruct((_NC, n, d), jnp.float32),
                  jax.ShapeDtypeStruct((_NC, n, 16), jnp.float32)),
        mesh=mesh,
        scratch_types=[] + [pltpu.VMEM((k,), jnp.int32)] * 9
          + [pltpu.VMEM((k,), jnp.float32)] * 6
          + [pltpu.VMEM((k, d), jnp.float32)] * 3
          + [pltpu.VMEM((k, 16), jnp.float32)] * 3
          + [pltpu.VMEM_SHARED((n, d), jnp.float32),
             pltpu.VMEM_SHARED((n, 16), jnp.float32)]
          + [pltpu.SemaphoreType.DMA] * 9,

        compiler_params=pltpu.CompilerParams(needs_layout_passes=False,
                                             use_tc_tiling_on_sc=False),
    )
    def sc_kernel(wh_hbm, edge_hbm, s1_hbm, s2_hbm,
                  onum_hbm, oden_hbm,
                  si0_v, si1_v, si2_v, di0_v, di1_v, di2_v,
                  dis0_v, dis1_v, dis2_v,
                  s1g0_v, s1g1_v, s1g2_v, s2g0_v, s2g1_v, s2g2_v,
                  rows0_v, rows1_v, rows2_v, pd0_v, pd1_v, pd2_v,
                  anum_sh, aden_sh,
                  gsem0, gsem1, gsem2, ssem0, ssem1, ssem2,
                  isem0, isem1, isem2):
        c = lax.axis_index("c")
        s = lax.axis_index("s")
        wid = c * _NS + s
        si_b = (si0_v, si1_v, si2_v)
        di_b = (di0_v, di1_v, di2_v)
        dis_b = (dis0_v, dis1_v, dis2_v)
        isem_b = (isem0, isem1, isem2)
        s1g_b = (s1g0_v, s1g1_v, s1g2_v)
        s2g_b = (s2g0_v, s2g1_v, s2g2_v)
        rows_b = (rows0_v, rows1_v, rows2_v)
        pd_b = (pd0_v, pd1_v, pd2_v)
        gsem_b = (gsem0, gsem1, gsem2)
        ssem_b = (ssem0, ssem1, ssem2)

        zero16 = jnp.zeros((16,), jnp.float32)

        def zrow(i, carry):
            for j in range(d // 16):
                rows0_v[i, pl.ds(j * 16, 16)] = zero16
            pd0_v[i, pl.ds(0, 16)] = zero16
            return carry

        lax.fori_loop(0, k, zrow, 0)

        nzc = n // zr

        def zacc(i, carry):
            cidx = i * _NS + s

            @pl.when(cidx < nzc)
            def _():
                pltpu.sync_copy(rows0_v, anum_sh.at[pl.ds(cidx * zr, zr)])
                pltpu.sync_copy(pd0_v, aden_sh.at[pl.ds(cidx * zr, zr)])

            return carry

        lax.fori_loop(0, -(-nzc // _NS), zacc, 0)
        plsc.subcore_barrier()

        lane0 = lax.iota(jnp.int32, 16) == 0
        ebase = wid * epw

        def _exp(t):
            a = jnp.clip(t, -80.0, 80.0) * 1.4426950408889634
            ni = (a + jnp.where(a >= 0.0, 0.5, -0.5)).astype(jnp.int32)
            r = a - ni.astype(jnp.float32)
            q = r * 0.6931471805599453
            pol = 1.0 + q * (1.0 + q * (0.5 + q * (
                0.16666667 + q * (0.041666668 + q * (
                    0.008333334 + q * 0.0013888889)))))
            scale = plsc.bitcast(lax.shift_left(ni + 127, 23), jnp.float32)
            return pol * scale

        def fire_idx(ci, b):
            pltpu.async_copy(edge_hbm.at[0, pl.ds(ebase + ci * k, k)],
                             si_b[b], isem_b[b])
            pltpu.async_copy(edge_hbm.at[1, pl.ds(ebase + ci * k, k)],
                             di_b[b], isem_b[b])

        def wait_idx(ci, b):
            pltpu.make_async_copy(edge_hbm.at[0, pl.ds(ebase + ci * k, k)],
                                  si_b[b], isem_b[b]).wait()
            pltpu.make_async_copy(edge_hbm.at[1, pl.ds(ebase + ci * k, k)],
                                  di_b[b], isem_b[b]).wait()

        def fire_gathers(ci, b):
            pltpu.async_copy(s1_hbm.at[si_b[b]], s1g_b[b], gsem_b[b])
            pltpu.async_copy(s2_hbm.at[di_b[b]], s2g_b[b], gsem_b[b])
            pltpu.async_copy(wh_hbm.at[si_b[b]], rows_b[b], gsem_b[b])

        def wait_gathers(b):
            pltpu.make_async_copy(s1_hbm.at[si_b[b]], s1g_b[b],
                                  gsem_b[b]).wait()
            pltpu.make_async_copy(s2_hbm.at[di_b[b]], s2g_b[b],
                                  gsem_b[b]).wait()
            pltpu.make_async_copy(wh_hbm.at[si_b[b]], rows_b[b],
                                  gsem_b[b]).wait()

        def fire_scatter(b):
            for g in range(k // 16):
                dis_b[b][pl.ds(g * 16, 16)] = di_b[b][pl.ds(g * 16, 16)]
            pltpu.async_copy(rows_b[b], anum_sh.at[dis_b[b]], ssem_b[b],
                             add=True)
            pltpu.async_copy(pd_b[b], aden_sh.at[dis_b[b]], ssem_b[b],
                             add=True)

        def wait_scatter(b):
            pltpu.make_async_copy(rows_b[b], anum_sh.at[dis_b[b]],
                                  ssem_b[b]).wait()
            pltpu.make_async_copy(pd_b[b], aden_sh.at[dis_b[b]],
                                  ssem_b[b]).wait()

        def compute(b):
            rows = rows_b[b]
            pd = pd_b[b]
            s1g = s1g_b[b]
            s2g = s2g_b[b]
            for g in range(k // 16):
                t = s1g[pl.ds(g * 16, 16)] + s2g[pl.ds(g * 16, 16)]
                t = jnp.where(t >= 0.0, t, 0.2 * t)
                s1g[pl.ds(g * 16, 16)] = _exp(t)

            def edge_group(gi, carry2):
                pg = s1g[pl.ds(gi * 16, 16)]
                for l in range(16):
                    ei = gi * 16 + l
                    pe = pg[l]
                    for j in range(d // 16):
                        rows[ei, pl.ds(j * 16, 16)] = (
                            rows[ei, pl.ds(j * 16, 16)] * pe)
                    pd[ei, pl.ds(0, 16)] = jnp.where(lane0, pe, 0.0)
                return carry2

            lax.fori_loop(0, k // 16, edge_group, 0)

        fire_idx(0, 0)
        fire_idx(1, 1)
        fire_idx(2, 2)
        wait_idx(0, 0)
        fire_gathers(0, 0)
        wait_idx(1, 1)
        fire_gathers(1, 1)

        def body(ci, t, first):
            b2 = (t + 2) % 3
            wait_gathers(t)
            compute(t)
            fire_scatter(t)

            @pl.when(ci + 3 < nchunk)
            def _():
                fire_idx(ci + 3, t)

            @pl.when(ci + 2 < nchunk)
            def _():
                wait_idx(ci + 2, b2)
                if not first:
                    wait_scatter(b2)
                fire_gathers(ci + 2, b2)

        def triple(j, carry):
            for t in range(3):
                body(3 * j + t, t, False)
            return carry

        body(0, 0, True)
        body(1, 1, False)
        body(2, 2, False)
        lax.fori_loop(1, nchunk // 3, triple, 0)
        wait_gathers(0)
        compute(0)
        fire_scatter(0)
        wait_gathers(1)
        compute(1)
        fire_scatter(1)
        wait_scatter(0)
        wait_scatter(1)
        wait_scatter(2)
        plsc.subcore_barrier()
        for kk in range(-(-nwc // _NS)):
            cidx = kk * _NS + s

            @pl.when(cidx < nwc)
            def _():
                pltpu.sync_copy(anum_sh.at[pl.ds(cidx * wr, wr)],
                                onum_hbm.at[c, pl.ds(cidx * wr, wr)])
                pltpu.sync_copy(aden_sh.at[pl.ds(cidx * wr, wr)],
                                oden_hbm.at[c, pl.ds(cidx * wr, wr)])

    return sc_kernel(wh, edge_index, s1, s2)


def _tc_fin_body(num_ref, den_ref, wh_ref, b_ref, o_ref):
    num = num_ref[0] + num_ref[1]
    den = den_ref[0, :, 0:1] + den_ref[1, :, 0:1]
    h = num / jnp.maximum(den, 1e-9) + wh_ref[...] + b_ref[...]
    o_ref[...] = jnp.maximum(h, 0.0)


def _tc_fin(acc_num, acc_den, wh, bias):
    n, d = wh.shape
    blk = 1000
    return pl.pallas_call(
        _tc_fin_body,
        grid=(n // blk,),
        in_specs=[
            pl.BlockSpec((_NC, blk, d), lambda i: (0, i, 0)),
            pl.BlockSpec((_NC, blk, 16), lambda i: (0, i, 0)),
            pl.BlockSpec((blk, d), lambda i: (i, 0)),
            pl.BlockSpec((1, d), lambda i: (0, 0)),
        ],
        out_specs=pl.BlockSpec((blk, d), lambda i: (i, 0)),
        out_shape=jax.ShapeDtypeStruct((n, d), jnp.float32),
    )(acc_num, acc_den, wh, bias)


def kernel(x, edge_index, W, a_w, bias):
    n = x.shape[0]
    wh, s1, s2 = _tc_prep(x, W, a_w)
    acc_num, acc_den = _sc_agg(wh, edge_index,
                               s1.reshape(n), s2.reshape(n))
    return _tc_fin(acc_num, acc_den, wh, bias)
